# baseline (speedup 1.0000x reference)
import functools

import jax
import jax.numpy as jnp
from jax.experimental import pallas as pl

NUM_SLOTS = 7
NEG_BIG = -3e38


def _slot_kernel(seq_ref, rank_ref, embed_ref, W1_ref, b1_ref, W2_ref, b2_ref,
                 g_ref, be_ref, Wq_ref, bq_ref, Wo_ref, bo_ref, out_ref):
    bB, L = seq_ref.shape
    V, H = embed_ref.shape

    f32 = jnp.float32
    dot = functools.partial(jax.lax.dot_general,
                            preferred_element_type=jnp.float32)

    E = embed_ref[...]
    h1v = jnp.maximum(dot(E, W1_ref[...], (((1,), (1,)), ((), ())))
                      + b1_ref[...], 0.0)
    ffv = dot(h1v, W2_ref[...], (((1,), (1,)), ((), ()))) + b2_ref[...]
    xv = E + ffv
    Jm = jnp.full((H, H), 1.0 / H, dtype=f32)
    xcv = xv - dot(xv, Jm, (((1,), (0,)), ((), ())))
    varb = dot(xcv * xcv, Jm, (((1,), (0,)), ((), ())))
    HSv = xcv * jax.lax.rsqrt(varb + 1e-5) * g_ref[...] + be_ref[...]
    HSW = dot(HSv, Wq_ref[...], (((1,), (1,)), ((), ())))

    seq = seq_ref[...][:, :, None]
    vocab_iota = jax.lax.broadcasted_iota(jnp.int32, (bB, L, V), 2)
    onehot = (seq == vocab_iota).astype(f32)
    oh_last = onehot[:, L - 1, :]

    rankp = jnp.sum(onehot * rank_ref[...][None], axis=2)
    col = jax.lax.broadcasted_iota(jnp.int32, (bB, L), 1).astype(f32)
    c = jnp.where(col < L - 3, rankp * 256.0 + col, 3e7)

    sel = jnp.zeros((bB, L), jnp.bool_)
    for _ in range(NUM_SLOTS):
        mval = jnp.min(c, axis=1, keepdims=True)
        pick = c == mval
        sel = jnp.logical_or(sel, pick)
        c = jnp.where(pick, 3e7, c)

    m = jnp.sum(onehot * sel.astype(f32)[:, :, None], axis=1)

    q = dot(oh_last, HSW, (((1,), (0,)), ((), ()))) + bq_ref[...]
    qlog = dot(q, HSv, (((1,), (1,)), ((), ()))) * (H ** -0.5)
    mx = jnp.max(jnp.where(m > 0.0, qlog, NEG_BIG), axis=1, keepdims=True)
    w = m * jnp.exp(jnp.minimum(qlog - mx, 0.0))
    wn = w / jnp.sum(w, axis=1, keepdims=True)
    ctx = dot(wn, HSv, (((1,), (0,)), ((), ())))

    out_ref[...] = dot(ctx, Wo_ref[...], (((1,), (1,)), ((), ()))) + bo_ref[...]


def kernel(seq, embed, W1, b1, W2, b2, gamma, beta, Wq, bq, Wo, bo):
    B, L = seq.shape
    V, H = embed.shape
    bB = min(256, B)
    grid = B // bB

    ev = embed
    ffv = jnp.maximum(ev @ W1.T + b1, 0.0) @ W2.T + b2
    xv = ev + ffv
    muv = jnp.mean(xv, axis=-1, keepdims=True)
    varv = jnp.var(xv, axis=-1, keepdims=True)
    hsv = (xv - muv) / jnp.sqrt(varv + 1e-5) * gamma + beta
    key = jnp.sqrt(jnp.sum(hsv * hsv, axis=-1))
    rank = jnp.sum((key[None, :] > key[:, None]).astype(jnp.float32),
                   axis=-1)

    row = lambda d: ((1, d), lambda i: (0, 0))
    specs = [
        pl.BlockSpec((bB, L), lambda i: (i, 0)),
        pl.BlockSpec(*row(V)),
        pl.BlockSpec((V, H), lambda i: (0, 0)),
        pl.BlockSpec((2 * H, H), lambda i: (0, 0)),
        pl.BlockSpec(*row(2 * H)),
        pl.BlockSpec((H, 2 * H), lambda i: (0, 0)),
        pl.BlockSpec(*row(H)),
        pl.BlockSpec(*row(H)),
        pl.BlockSpec(*row(H)),
        pl.BlockSpec((H, H), lambda i: (0, 0)),
        pl.BlockSpec(*row(H)),
        pl.BlockSpec((V, H), lambda i: (0, 0)),
        pl.BlockSpec(*row(V)),
    ]

    return pl.pallas_call(
        _slot_kernel,
        grid=(grid,),
        in_specs=specs,
        out_specs=pl.BlockSpec((bB, V), lambda i: (i, 0)),
        out_shape=jax.ShapeDtypeStruct((B, V), jnp.float32),
    )(seq.astype(jnp.int32), rank.reshape(1, -1), embed, W1,
      b1.reshape(1, -1), W2, b2.reshape(1, -1), gamma.reshape(1, -1),
      beta.reshape(1, -1), Wq, bq.reshape(1, -1), Wo, bo.reshape(1, -1))


# speedup vs baseline: 1.2656x; 1.2656x over previous
import functools

import jax
import jax.numpy as jnp
from jax.experimental import pallas as pl

NUM_SLOTS = 7
NEG_BIG = -3e38


def _slot_kernel(seq_ref, key_ref, embed_ref, W1_ref, b1_ref, W2_ref, b2_ref,
                 g_ref, be_ref, Wq_ref, bq_ref, Wo_ref, bo_ref, out_ref):
    bB, L = seq_ref.shape
    V, H = embed_ref.shape

    f32 = jnp.float32
    dot = functools.partial(jax.lax.dot_general,
                            preferred_element_type=jnp.float32)

    E = embed_ref[...]
    h1v = jnp.maximum(dot(E, W1_ref[...], (((1,), (1,)), ((), ())))
                      + b1_ref[...], 0.0)
    ffv = dot(h1v, W2_ref[...], (((1,), (1,)), ((), ()))) + b2_ref[...]
    xv = E + ffv
    Jm = jnp.full((H, H), 1.0 / H, dtype=f32)
    xcv = xv - dot(xv, Jm, (((1,), (0,)), ((), ())))
    varb = dot(xcv * xcv, Jm, (((1,), (0,)), ((), ())))
    HSv = xcv * jax.lax.rsqrt(varb + 1e-5) * g_ref[...] + be_ref[...]
    HSW = dot(HSv, Wq_ref[...], (((1,), (1,)), ((), ())))

    seq = seq_ref[...][:, :, None]
    vocab_iota = jax.lax.broadcasted_iota(jnp.int32, (bB, L, V), 2)
    onehot = (seq == vocab_iota).astype(f32)
    oh_last = onehot[:, L - 1, :]

    keyp = jnp.sum(onehot * key_ref[...][None], axis=2)
    col = jax.lax.broadcasted_iota(jnp.int32, (bB, L), 1)
    v = jnp.where(col < L - 3, keyp, NEG_BIG)

    sel = jnp.zeros((bB, L), jnp.bool_)
    for _ in range(NUM_SLOTS):
        mval = jnp.max(v, axis=1, keepdims=True)
        is_max = v == mval
        first = jnp.min(jnp.where(is_max, col, L), axis=1, keepdims=True)
        pick = col == first
        sel = jnp.logical_or(sel, pick)
        v = jnp.where(pick, NEG_BIG, v)

    m = jnp.sum(onehot * sel.astype(f32)[:, :, None], axis=1)

    q = dot(oh_last, HSW, (((1,), (0,)), ((), ()))) + bq_ref[...]
    qlog = dot(q, HSv, (((1,), (1,)), ((), ()))) * (H ** -0.5)
    mx = jnp.max(jnp.where(m > 0.0, qlog, NEG_BIG), axis=1, keepdims=True)
    w = m * jnp.exp(jnp.minimum(qlog - mx, 0.0))
    wn = w / jnp.sum(w, axis=1, keepdims=True)
    ctx = dot(wn, HSv, (((1,), (0,)), ((), ())))

    out_ref[...] = dot(ctx, Wo_ref[...], (((1,), (1,)), ((), ()))) + bo_ref[...]


def kernel(seq, embed, W1, b1, W2, b2, gamma, beta, Wq, bq, Wo, bo):
    B, L = seq.shape
    V, H = embed.shape
    bB = min(256, B)
    grid = B // bB

    ev = embed
    ffv = jnp.maximum(ev @ W1.T + b1, 0.0) @ W2.T + b2
    xv = ev + ffv
    muv = jnp.mean(xv, axis=-1, keepdims=True)
    varv = jnp.var(xv, axis=-1, keepdims=True)
    hsv = (xv - muv) / jnp.sqrt(varv + 1e-5) * gamma + beta
    key = jnp.sqrt(jnp.sum(hsv * hsv, axis=-1))

    row = lambda d: ((1, d), lambda i: (0, 0))
    specs = [
        pl.BlockSpec((bB, L), lambda i: (i, 0)),
        pl.BlockSpec(*row(V)),
        pl.BlockSpec((V, H), lambda i: (0, 0)),
        pl.BlockSpec((2 * H, H), lambda i: (0, 0)),
        pl.BlockSpec(*row(2 * H)),
        pl.BlockSpec((H, 2 * H), lambda i: (0, 0)),
        pl.BlockSpec(*row(H)),
        pl.BlockSpec(*row(H)),
        pl.BlockSpec(*row(H)),
        pl.BlockSpec((H, H), lambda i: (0, 0)),
        pl.BlockSpec(*row(H)),
        pl.BlockSpec((V, H), lambda i: (0, 0)),
        pl.BlockSpec(*row(V)),
    ]

    return pl.pallas_call(
        _slot_kernel,
        grid=(grid,),
        in_specs=specs,
        out_specs=pl.BlockSpec((bB, V), lambda i: (i, 0)),
        out_shape=jax.ShapeDtypeStruct((B, V), jnp.float32),
    )(seq.astype(jnp.int32), key.reshape(1, -1), embed, W1,
      b1.reshape(1, -1), W2, b2.reshape(1, -1), gamma.reshape(1, -1),
      beta.reshape(1, -1), Wq, bq.reshape(1, -1), Wo, bo.reshape(1, -1))
